# Initial kernel scaffold; baseline (speedup 1.0000x reference)
import jax
import jax.numpy as jnp
from jax.experimental import pallas as pl


def kernel(x, edge_index, Wl0, bl0, Wr0, Wl1, bl1, Wr1, Wl2, bl2, Wr2, Wc, bc):
    raise NotImplementedError("write your pallas kernel here")


# speedup vs baseline: 6.1281x; 6.1281x over previous
import functools

import jax
import jax.numpy as jnp
from jax import lax
from jax.experimental import pallas as pl
from jax.experimental.pallas import tpu as pltpu
from jax.experimental.pallas import tpu_sc as plsc

N = 10000
E = 320000
F = 128
FH = F // 2
NCLS = 64

NCORES = 2
NSUB = 16
EPS = E // NSUB
C = 125
NCHUNK = EPS // C
Z = 200
NZ = N // Z
CW = 16

_mesh = plsc.VectorSubcoreMesh(core_axis_name="c", subcore_axis_name="s")


def _zero_vmem(ref, nrow, ncol):
    def body(i, carry):
        ref[i // (ncol // 16), pl.ds((i % (ncol // 16)) * 16, 16)] = (
            jnp.zeros((16,), jnp.float32))
        return carry
    lax.fori_loop(0, nrow * (ncol // 16), body, None)


def _sc_body(h_hbm, src_hbm, dst_hbm, out_hbm, cnt_hbm, idx_s, idx_d, rows,
             ones, zbuf, zbuf16, acc, cnt_sh, sem, *, with_cnt):
    c = lax.axis_index("c")
    s = lax.axis_index("s")

    pltpu.sync_copy(src_hbm.at[c, s], idx_s)
    pltpu.sync_copy(dst_hbm.at[s], idx_d)

    _zero_vmem(zbuf, Z, FH)
    if with_cnt:
        _zero_vmem(zbuf16, Z, CW)

        def fill_ones(i, carry):
            ones[i, pl.ds(0, CW)] = jnp.ones((CW,), jnp.float32)
            return carry
        lax.fori_loop(0, C, fill_ones, None)
    for t in range((NZ + NSUB - 1) // NSUB):
        j = s + NSUB * t

        @pl.when(j < NZ)
        def _():
            pltpu.sync_copy(zbuf, acc.at[pl.ds(j * Z, Z)])
            if with_cnt:
                pltpu.sync_copy(zbuf16, cnt_sh.at[pl.ds(j * Z, Z)])
    plsc.subcore_barrier()

    def chunk(k, carry):
        pltpu.async_copy(h_hbm.at[idx_s.at[k]], rows, sem).wait()
        pltpu.sync_copy(rows, acc.at[idx_d.at[k]], add=True)
        if with_cnt:
            @pl.when((k < NCHUNK // 2) == (c == 0))
            def _():
                pltpu.sync_copy(ones, cnt_sh.at[idx_d.at[k]], add=True)
        return carry
    lax.fori_loop(0, NCHUNK, chunk, None)

    plsc.subcore_barrier()
    for t in range((NZ + NSUB - 1) // NSUB):
        j = s + NSUB * t

        @pl.when(j < NZ)
        def _():
            pltpu.sync_copy(acc.at[pl.ds(j * Z, Z)],
                            out_hbm.at[c, pl.ds(j * Z, Z)])
            if with_cnt:
                pltpu.sync_copy(cnt_sh.at[pl.ds(j * Z, Z)],
                                cnt_hbm.at[c, pl.ds(j * Z, Z)])


def _make_sc_spmm(with_cnt):
    acc_type = jax.ShapeDtypeStruct((NCORES, N, FH), jnp.float32)
    scratch = [
        pltpu.VMEM((NCHUNK, C), jnp.int32),
        pltpu.VMEM((NCHUNK, C), jnp.int32),
        pltpu.VMEM((C, FH), jnp.float32),
        pltpu.VMEM((C, CW), jnp.float32),
        pltpu.VMEM((Z, FH), jnp.float32),
        pltpu.VMEM((Z, CW), jnp.float32),
        pltpu.VMEM_SHARED((N, FH), jnp.float32),
    ]
    if with_cnt:
        scratch.append(pltpu.VMEM_SHARED((N, CW), jnp.float32))
        out_type = (acc_type, jax.ShapeDtypeStruct((NCORES, N, CW),
                                                   jnp.float32))

        def body(h, src, dst, out, cnt, idx_s, idx_d, rows, ones, zbuf,
                 zbuf16, acc, cnt_sh, sem):
            _sc_body(h, src, dst, out, cnt, idx_s, idx_d, rows, ones, zbuf,
                     zbuf16, acc, cnt_sh, sem, with_cnt=True)
    else:
        out_type = acc_type

        def body(h, src, dst, out, idx_s, idx_d, rows, ones, zbuf, zbuf16,
                 acc, sem):
            _sc_body(h, src, dst, out, None, idx_s, idx_d, rows, ones, zbuf,
                     zbuf16, acc, None, sem, with_cnt=False)
    scratch.append(pltpu.SemaphoreType.DMA)
    return pl.kernel(body, out_type=out_type, mesh=_mesh,
                     scratch_types=scratch,
                     compiler_params=pltpu.CompilerParams(
                         use_tc_tiling_on_sc=False))


_sc_spmm_cnt = _make_sc_spmm(True)
_sc_spmm = _make_sc_spmm(False)

R = 1000


def _dots(mean, h, wl_ref, wr_ref, bl_ref):
    return (jnp.dot(mean, wl_ref[...], preferred_element_type=jnp.float32,
                    precision=lax.Precision.HIGHEST)
            + jnp.dot(h, wr_ref[...], preferred_element_type=jnp.float32,
                      precision=lax.Precision.HIGHEST)
            + bl_ref[...])


def _combine_body(p_ref, c_ref, h_ref, wl_ref, wr_ref, bl_ref, o_ref):
    cnt = jnp.maximum(c_ref[0, :, 0:1] + c_ref[1, :, 0:1], 1.0)
    mean = jnp.concatenate([p_ref[0], p_ref[1]], axis=1) / cnt
    h = jnp.concatenate([h_ref[0], h_ref[1]], axis=1)
    y = jnp.maximum(_dots(mean, h, wl_ref, wr_ref, bl_ref), 0.0)
    o_ref[0] = y[:, :FH]
    o_ref[1] = y[:, FH:]


def _final_body(p_ref, c_ref, h_ref, wl_ref, wr_ref, bl_ref, wc_ref, bc_ref,
                o_ref):
    cnt = jnp.maximum(c_ref[0, :, 0:1] + c_ref[1, :, 0:1], 1.0)
    mean = jnp.concatenate([p_ref[0], p_ref[1]], axis=1) / cnt
    h = jnp.concatenate([h_ref[0], h_ref[1]], axis=1)
    y = _dots(mean, h, wl_ref, wr_ref, bl_ref)
    o_ref[...] = (jnp.dot(y, wc_ref[...], preferred_element_type=jnp.float32,
                          precision=lax.Precision.HIGHEST)
                  + bc_ref[...])


_common_specs = [
    pl.BlockSpec((NCORES, R, FH), lambda i: (0, i, 0)),
    pl.BlockSpec((NCORES, R, CW), lambda i: (0, i, 0)),
    pl.BlockSpec((NCORES, R, FH), lambda i: (0, i, 0)),
    pl.BlockSpec((F, F), lambda i: (0, 0)),
    pl.BlockSpec((F, F), lambda i: (0, 0)),
    pl.BlockSpec((1, F), lambda i: (0, 0)),
]

_combine_relu = pl.pallas_call(
    _combine_body,
    grid=(N // R,),
    in_specs=_common_specs,
    out_specs=pl.BlockSpec((NCORES, R, FH), lambda i: (0, i, 0)),
    out_shape=jax.ShapeDtypeStruct((NCORES, N, FH), jnp.float32),
)

_combine_final = pl.pallas_call(
    _final_body,
    grid=(N // R,),
    in_specs=_common_specs + [
        pl.BlockSpec((F, NCLS), lambda i: (0, 0)),
        pl.BlockSpec((1, NCLS), lambda i: (0, 0)),
    ],
    out_specs=pl.BlockSpec((R, NCLS), lambda i: (i, 0)),
    out_shape=jax.ShapeDtypeStruct((N, NCLS), jnp.float32),
)


def kernel(x, edge_index, Wl0, bl0, Wr0, Wl1, bl1, Wr1, Wl2, bl2, Wr2, Wc,
           bc):
    src = edge_index[0].reshape(NSUB, NCHUNK, C)
    dst = edge_index[1].reshape(NSUB, NCHUNK, C)
    src2 = jnp.stack([src, src + N])
    x2 = jnp.stack([x[:, :FH], x[:, FH:]])

    p0, cnt = _sc_spmm_cnt(x2.reshape(NCORES * N, FH), src2, dst)
    h1 = _combine_relu(p0, cnt, x2, Wl0, Wr0, bl0.reshape(1, F))
    p1 = _sc_spmm(h1.reshape(NCORES * N, FH), src2, dst)
    h2 = _combine_relu(p1, cnt, h1, Wl1, Wr1, bl1.reshape(1, F))
    p2 = _sc_spmm(h2.reshape(NCORES * N, FH), src2, dst)
    return _combine_final(p2, cnt, h2, Wl2, Wr2, bl2.reshape(1, F), Wc,
                          bc.reshape(1, NCLS))


# speedup vs baseline: 9.1402x; 1.4915x over previous
import functools

import jax
import jax.numpy as jnp
from jax import lax
from jax.experimental import pallas as pl
from jax.experimental.pallas import tpu as pltpu
from jax.experimental.pallas import tpu_sc as plsc

N = 10000
E = 320000
F = 128
FH = F // 2
NCLS = 64

NCORES = 2
NSUB = 16
EPS = E // NSUB
C = 125
NCHUNK = EPS // C
Z = 200
NZ = N // Z
CW = 16

_mesh = plsc.VectorSubcoreMesh(core_axis_name="c", subcore_axis_name="s")


def _zero_vmem(ref, nrow, ncol):
    def body(i, carry):
        ref[i // (ncol // 16), pl.ds((i % (ncol // 16)) * 16, 16)] = (
            jnp.zeros((16,), jnp.float32))
        return carry
    lax.fori_loop(0, nrow * (ncol // 16), body, None)


def _sc_body(h_hbm, src_hbm, dst_hbm, out_hbm, cnt_hbm, idx_s, idx_d, rows0,
             rows1, ones, zbuf, zbuf16, acc, cnt_sh, sem_g, sem_s, sem_c, *,
             with_cnt):
    c = lax.axis_index("c")
    s = lax.axis_index("s")

    pltpu.sync_copy(src_hbm.at[c, s], idx_s)
    pltpu.sync_copy(dst_hbm.at[s], idx_d)

    _zero_vmem(zbuf, Z, FH)
    if with_cnt:
        _zero_vmem(zbuf16, Z, CW)

        def fill_ones(i, carry):
            ones[i, pl.ds(0, CW)] = jnp.ones((CW,), jnp.float32)
            return carry
        lax.fori_loop(0, C, fill_ones, None)
    for t in range((NZ + NSUB - 1) // NSUB):
        j = s + NSUB * t

        @pl.when(j < NZ)
        def _():
            pltpu.sync_copy(zbuf, acc.at[pl.ds(j * Z, Z)])
            if with_cnt:
                pltpu.sync_copy(zbuf16, cnt_sh.at[pl.ds(j * Z, Z)])
    plsc.subcore_barrier()

    pltpu.async_copy(h_hbm.at[idx_s.at[0]], rows0, sem_g)
    pltpu.async_copy(h_hbm.at[idx_s.at[1]], rows1, sem_g)

    def chunk(i, carry):
        for b, rows in ((0, rows0), (1, rows1)):
            k = 2 * i + b
            pltpu.make_async_copy(h_hbm.at[idx_s.at[k]], rows, sem_g).wait()
            pltpu.async_copy(rows, acc.at[idx_d.at[k]], sem_s, add=True)
            if with_cnt:
                @pl.when((k < NCHUNK // 2) == (c == 0))
                def _():
                    pltpu.async_copy(ones, cnt_sh.at[idx_d.at[k]], sem_c,
                                     add=True)
        for b, rows in ((0, rows0), (1, rows1)):
            k = 2 * i + b
            pltpu.make_async_copy(rows, acc.at[idx_d.at[k]], sem_s).wait()

            @pl.when(k + 2 < NCHUNK)
            def _():
                pltpu.async_copy(h_hbm.at[idx_s.at[k + 2]], rows, sem_g)
        return carry
    lax.fori_loop(0, NCHUNK // 2, chunk, None)
    if with_cnt:
        def drain(i, carry):
            pltpu.make_async_copy(ones, cnt_sh.at[idx_d.at[0]], sem_c).wait()
            return carry
        lax.fori_loop(0, NCHUNK // 2, drain, None)

    plsc.subcore_barrier()
    for t in range((NZ + NSUB - 1) // NSUB):
        j = s + NSUB * t

        @pl.when(j < NZ)
        def _():
            pltpu.sync_copy(acc.at[pl.ds(j * Z, Z)],
                            out_hbm.at[c, pl.ds(j * Z, Z)])
            if with_cnt:
                pltpu.sync_copy(cnt_sh.at[pl.ds(j * Z, Z)],
                                cnt_hbm.at[c, pl.ds(j * Z, Z)])


def _make_sc_spmm(with_cnt):
    acc_type = jax.ShapeDtypeStruct((NCORES, N, FH), jnp.float32)
    scratch = [
        pltpu.VMEM((NCHUNK, C), jnp.int32),
        pltpu.VMEM((NCHUNK, C), jnp.int32),
        pltpu.VMEM((C, FH), jnp.float32),
        pltpu.VMEM((C, FH), jnp.float32),
        pltpu.VMEM((C, CW), jnp.float32),
        pltpu.VMEM((Z, FH), jnp.float32),
        pltpu.VMEM((Z, CW), jnp.float32),
        pltpu.VMEM_SHARED((N, FH), jnp.float32),
    ]
    if with_cnt:
        scratch.append(pltpu.VMEM_SHARED((N, CW), jnp.float32))
        out_type = (acc_type, jax.ShapeDtypeStruct((NCORES, N, CW),
                                                   jnp.float32))

        def body(h, src, dst, out, cnt, idx_s, idx_d, rows0, rows1, ones,
                 zbuf, zbuf16, acc, cnt_sh, sem_g, sem_s, sem_c):
            _sc_body(h, src, dst, out, cnt, idx_s, idx_d, rows0, rows1, ones,
                     zbuf, zbuf16, acc, cnt_sh, sem_g, sem_s, sem_c,
                     with_cnt=True)
    else:
        out_type = acc_type

        def body(h, src, dst, out, idx_s, idx_d, rows0, rows1, ones, zbuf,
                 zbuf16, acc, sem_g, sem_s, sem_c):
            _sc_body(h, src, dst, out, None, idx_s, idx_d, rows0, rows1, ones,
                     zbuf, zbuf16, acc, None, sem_g, sem_s, sem_c,
                     with_cnt=False)
    scratch.extend([pltpu.SemaphoreType.DMA, pltpu.SemaphoreType.DMA,
                    pltpu.SemaphoreType.DMA])
    return pl.kernel(body, out_type=out_type, mesh=_mesh,
                     scratch_types=scratch,
                     compiler_params=pltpu.CompilerParams(
                         use_tc_tiling_on_sc=False))


_sc_spmm_cnt = _make_sc_spmm(True)
_sc_spmm = _make_sc_spmm(False)

R = 1000


def _dots(mean, h, wl_ref, wr_ref, bl_ref):
    return (jnp.dot(mean, wl_ref[...], preferred_element_type=jnp.float32,
                    precision=lax.Precision.HIGHEST)
            + jnp.dot(h, wr_ref[...], preferred_element_type=jnp.float32,
                      precision=lax.Precision.HIGHEST)
            + bl_ref[...])


def _combine_body(p_ref, c_ref, h_ref, wl_ref, wr_ref, bl_ref, o_ref):
    cnt = jnp.maximum(c_ref[0, :, 0:1] + c_ref[1, :, 0:1], 1.0)
    mean = jnp.concatenate([p_ref[0], p_ref[1]], axis=1) / cnt
    h = jnp.concatenate([h_ref[0], h_ref[1]], axis=1)
    y = jnp.maximum(_dots(mean, h, wl_ref, wr_ref, bl_ref), 0.0)
    o_ref[0] = y[:, :FH]
    o_ref[1] = y[:, FH:]


def _final_body(p_ref, c_ref, h_ref, wl_ref, wr_ref, bl_ref, wc_ref, bc_ref,
                o_ref):
    cnt = jnp.maximum(c_ref[0, :, 0:1] + c_ref[1, :, 0:1], 1.0)
    mean = jnp.concatenate([p_ref[0], p_ref[1]], axis=1) / cnt
    h = jnp.concatenate([h_ref[0], h_ref[1]], axis=1)
    y = _dots(mean, h, wl_ref, wr_ref, bl_ref)
    o_ref[...] = (jnp.dot(y, wc_ref[...], preferred_element_type=jnp.float32,
                          precision=lax.Precision.HIGHEST)
                  + bc_ref[...])


_common_specs = [
    pl.BlockSpec((NCORES, R, FH), lambda i: (0, i, 0)),
    pl.BlockSpec((NCORES, R, CW), lambda i: (0, i, 0)),
    pl.BlockSpec((NCORES, R, FH), lambda i: (0, i, 0)),
    pl.BlockSpec((F, F), lambda i: (0, 0)),
    pl.BlockSpec((F, F), lambda i: (0, 0)),
    pl.BlockSpec((1, F), lambda i: (0, 0)),
]

_combine_relu = pl.pallas_call(
    _combine_body,
    grid=(N // R,),
    in_specs=_common_specs,
    out_specs=pl.BlockSpec((NCORES, R, FH), lambda i: (0, i, 0)),
    out_shape=jax.ShapeDtypeStruct((NCORES, N, FH), jnp.float32),
)

_combine_final = pl.pallas_call(
    _final_body,
    grid=(N // R,),
    in_specs=_common_specs + [
        pl.BlockSpec((F, NCLS), lambda i: (0, 0)),
        pl.BlockSpec((1, NCLS), lambda i: (0, 0)),
    ],
    out_specs=pl.BlockSpec((R, NCLS), lambda i: (i, 0)),
    out_shape=jax.ShapeDtypeStruct((N, NCLS), jnp.float32),
)


def kernel(x, edge_index, Wl0, bl0, Wr0, Wl1, bl1, Wr1, Wl2, bl2, Wr2, Wc,
           bc):
    src = edge_index[0].reshape(NSUB, NCHUNK, C)
    dst = edge_index[1].reshape(NSUB, NCHUNK, C)
    src2 = jnp.stack([src, src + N])
    x2 = jnp.stack([x[:, :FH], x[:, FH:]])

    p0, cnt = _sc_spmm_cnt(x2.reshape(NCORES * N, FH), src2, dst)
    h1 = _combine_relu(p0, cnt, x2, Wl0, Wr0, bl0.reshape(1, F))
    p1 = _sc_spmm(h1.reshape(NCORES * N, FH), src2, dst)
    h2 = _combine_relu(p1, cnt, h1, Wl1, Wr1, bl1.reshape(1, F))
    p2 = _sc_spmm(h2.reshape(NCORES * N, FH), src2, dst)
    return _combine_final(p2, cnt, h2, Wl2, Wr2, bl2.reshape(1, F), Wc,
                          bc.reshape(1, NCLS))


# speedup vs baseline: 11.1843x; 1.2236x over previous
import functools

import jax
import jax.numpy as jnp
from jax import lax
from jax.experimental import pallas as pl
from jax.experimental.pallas import tpu as pltpu
from jax.experimental.pallas import tpu_sc as plsc

N = 10000
E = 320000
F = 128
FH = F // 2
NCLS = 64

NCORES = 2
NSUB = 16
EPS = E // NSUB
C = 125
NCHUNK = EPS // C
Z = 200
NZ = N // Z
CW = 16

_mesh = plsc.VectorSubcoreMesh(core_axis_name="c", subcore_axis_name="s")


def _zero_vmem(ref, nrow, ncol):
    def body(i, carry):
        ref[i // (ncol // 16), pl.ds((i % (ncol // 16)) * 16, 16)] = (
            jnp.zeros((16,), jnp.float32))
        return carry
    lax.fori_loop(0, nrow * (ncol // 16), body, None)


NBUF = 4


def _spmm_body(h_hbm, src_hbm, dst_hbm, out_hbm, idx_s, idx_d, bufs, zbuf,
               acc, sem_g, sem_s):
    c = lax.axis_index("c")
    s = lax.axis_index("s")

    pltpu.sync_copy(src_hbm.at[c, s], idx_s)
    pltpu.sync_copy(dst_hbm.at[s], idx_d)

    _zero_vmem(zbuf, Z, FH)
    for t in range((NZ + NSUB - 1) // NSUB):
        j = s + NSUB * t

        @pl.when(j < NZ)
        def _():
            pltpu.sync_copy(zbuf, acc.at[pl.ds(j * Z, Z)])
    plsc.subcore_barrier()

    for b in range(NBUF):
        pltpu.async_copy(h_hbm.at[idx_s.at[b]], bufs[b], sem_g)

    def chunk(i, carry):
        for b in range(NBUF):
            k = NBUF * i + b
            pltpu.make_async_copy(h_hbm.at[idx_s.at[k]], bufs[b],
                                  sem_g).wait()
            pltpu.async_copy(bufs[b], acc.at[idx_d.at[k]], sem_s, add=True)
        for b in range(NBUF):
            k = NBUF * i + b
            pltpu.make_async_copy(bufs[b], acc.at[idx_d.at[k]], sem_s).wait()

            @pl.when(k + NBUF < NCHUNK)
            def _():
                pltpu.async_copy(h_hbm.at[idx_s.at[k + NBUF]], bufs[b], sem_g)
        return carry
    lax.fori_loop(0, NCHUNK // NBUF, chunk, None)

    plsc.subcore_barrier()
    for t in range((NZ + NSUB - 1) // NSUB):
        j = s + NSUB * t

        @pl.when(j < NZ)
        def _():
            pltpu.sync_copy(acc.at[pl.ds(j * Z, Z)],
                            out_hbm.at[c, pl.ds(j * Z, Z)])


_sc_spmm = pl.kernel(
    _spmm_body,
    out_type=jax.ShapeDtypeStruct((NCORES, N, FH), jnp.float32),
    mesh=_mesh,
    scratch_types=[
        pltpu.VMEM((NCHUNK, C), jnp.int32),
        pltpu.VMEM((NCHUNK, C), jnp.int32),
        tuple(pltpu.VMEM((C, FH), jnp.float32) for _ in range(NBUF)),
        pltpu.VMEM((Z, FH), jnp.float32),
        pltpu.VMEM_SHARED((N, FH), jnp.float32),
        pltpu.SemaphoreType.DMA,
        pltpu.SemaphoreType.DMA,
    ],
    compiler_params=pltpu.CompilerParams(use_tc_tiling_on_sc=False))

NCHUNK_D = (E // (NCORES * NSUB)) // C
WIN = 8


def _cnt_body(dst_hbm, cnt_hbm, idx_d, ones, zbuf16, cnt_sh, sem):
    c = lax.axis_index("c")
    s = lax.axis_index("s")
    wid = c * NSUB + s
    pltpu.sync_copy(dst_hbm.at[wid], idx_d)

    _zero_vmem(zbuf16, Z, CW)

    def fill_ones(i, carry):
        ones[i, pl.ds(0, CW)] = jnp.ones((CW,), jnp.float32)
        return carry
    lax.fori_loop(0, C, fill_ones, None)
    for t in range((NZ + NSUB - 1) // NSUB):
        j = s + NSUB * t

        @pl.when(j < NZ)
        def _():
            pltpu.sync_copy(zbuf16, cnt_sh.at[pl.ds(j * Z, Z)])
    plsc.subcore_barrier()

    def chunk(k, carry):
        pltpu.async_copy(ones, cnt_sh.at[idx_d.at[k]], sem, add=True)

        @pl.when(k >= WIN)
        def _():
            pltpu.make_async_copy(ones, cnt_sh.at[idx_d.at[0]], sem).wait()
        return carry
    lax.fori_loop(0, NCHUNK_D, chunk, None)
    for _ in range(WIN):
        pltpu.make_async_copy(ones, cnt_sh.at[idx_d.at[0]], sem).wait()

    plsc.subcore_barrier()
    for t in range((NZ + NSUB - 1) // NSUB):
        j = s + NSUB * t

        @pl.when(j < NZ)
        def _():
            pltpu.sync_copy(cnt_sh.at[pl.ds(j * Z, Z)],
                            cnt_hbm.at[c, pl.ds(j * Z, Z)])


_sc_cnt = pl.kernel(
    _cnt_body,
    out_type=jax.ShapeDtypeStruct((NCORES, N, CW), jnp.float32),
    mesh=_mesh,
    scratch_types=[
        pltpu.VMEM((NCHUNK_D, C), jnp.int32),
        pltpu.VMEM((C, CW), jnp.float32),
        pltpu.VMEM((Z, CW), jnp.float32),
        pltpu.VMEM_SHARED((N, CW), jnp.float32),
        pltpu.SemaphoreType.DMA,
    ],
    compiler_params=pltpu.CompilerParams(use_tc_tiling_on_sc=False))

R = 2000


def _dots(mean, h, wl_ref, wr_ref, bl_ref):
    return (jnp.dot(mean, wl_ref[...], preferred_element_type=jnp.float32,
                    precision=lax.Precision.DEFAULT)
            + jnp.dot(h, wr_ref[...], preferred_element_type=jnp.float32,
                      precision=lax.Precision.DEFAULT)
            + bl_ref[...])


def _combine_body(p_ref, c_ref, h_ref, wl_ref, wr_ref, bl_ref, o_ref):
    cnt = jnp.maximum(c_ref[0, :, 0:1] + c_ref[1, :, 0:1], 1.0)
    mean = jnp.concatenate([p_ref[0], p_ref[1]], axis=1) / cnt
    h = jnp.concatenate([h_ref[0], h_ref[1]], axis=1)
    y = jnp.maximum(_dots(mean, h, wl_ref, wr_ref, bl_ref), 0.0)
    o_ref[0] = y[:, :FH]
    o_ref[1] = y[:, FH:]


def _final_body(p_ref, c_ref, h_ref, wl_ref, wr_ref, bl_ref, wc_ref, bc_ref,
                o_ref):
    cnt = jnp.maximum(c_ref[0, :, 0:1] + c_ref[1, :, 0:1], 1.0)
    mean = jnp.concatenate([p_ref[0], p_ref[1]], axis=1) / cnt
    h = jnp.concatenate([h_ref[0], h_ref[1]], axis=1)
    y = _dots(mean, h, wl_ref, wr_ref, bl_ref)
    o_ref[...] = (jnp.dot(y, wc_ref[...], preferred_element_type=jnp.float32,
                          precision=lax.Precision.DEFAULT)
                  + bc_ref[...])


_common_specs = [
    pl.BlockSpec((NCORES, R, FH), lambda i: (0, i, 0)),
    pl.BlockSpec((NCORES, R, CW), lambda i: (0, i, 0)),
    pl.BlockSpec((NCORES, R, FH), lambda i: (0, i, 0)),
    pl.BlockSpec((F, F), lambda i: (0, 0)),
    pl.BlockSpec((F, F), lambda i: (0, 0)),
    pl.BlockSpec((1, F), lambda i: (0, 0)),
]

_combine_relu = pl.pallas_call(
    _combine_body,
    grid=(N // R,),
    in_specs=_common_specs,
    out_specs=pl.BlockSpec((NCORES, R, FH), lambda i: (0, i, 0)),
    out_shape=jax.ShapeDtypeStruct((NCORES, N, FH), jnp.float32),
)

_combine_final = pl.pallas_call(
    _final_body,
    grid=(N // R,),
    in_specs=_common_specs + [
        pl.BlockSpec((F, NCLS), lambda i: (0, 0)),
        pl.BlockSpec((1, NCLS), lambda i: (0, 0)),
    ],
    out_specs=pl.BlockSpec((R, NCLS), lambda i: (i, 0)),
    out_shape=jax.ShapeDtypeStruct((N, NCLS), jnp.float32),
)


def kernel(x, edge_index, Wl0, bl0, Wr0, Wl1, bl1, Wr1, Wl2, bl2, Wr2, Wc,
           bc):
    src = edge_index[0].reshape(NSUB, NCHUNK, C)
    dst = edge_index[1].reshape(NSUB, NCHUNK, C)
    src2 = jnp.stack([src, src + N])
    dst_w = edge_index[1].reshape(NCORES * NSUB, NCHUNK_D, C)
    x2 = jnp.stack([x[:, :FH], x[:, FH:]])

    cnt = _sc_cnt(dst_w)
    p0 = _sc_spmm(x2.reshape(NCORES * N, FH), src2, dst)
    h1 = _combine_relu(p0, cnt, x2, Wl0, Wr0, bl0.reshape(1, F))
    p1 = _sc_spmm(h1.reshape(NCORES * N, FH), src2, dst)
    h2 = _combine_relu(p1, cnt, h1, Wl1, Wr1, bl1.reshape(1, F))
    p2 = _sc_spmm(h2.reshape(NCORES * N, FH), src2, dst)
    return _combine_final(p2, cnt, h2, Wl2, Wr2, bl2.reshape(1, F), Wc,
                          bc.reshape(1, NCLS))


# speedup vs baseline: 11.3024x; 1.0106x over previous
import functools

import jax
import jax.numpy as jnp
from jax import lax
from jax.experimental import pallas as pl
from jax.experimental.pallas import tpu as pltpu
from jax.experimental.pallas import tpu_sc as plsc

N = 10000
E = 320000
F = 128
FH = F // 2
NCLS = 64

NCORES = 2
NSUB = 16
EPS = E // NSUB
C = 125
NCHUNK = EPS // C
Z = 200
NZ = N // Z
CW = 16

_mesh = plsc.VectorSubcoreMesh(core_axis_name="c", subcore_axis_name="s")


def _zero_vmem(ref, nrow, ncol):
    def body(i, carry):
        ref[i // (ncol // 16), pl.ds((i % (ncol // 16)) * 16, 16)] = (
            jnp.zeros((16,), jnp.float32))
        return carry
    lax.fori_loop(0, nrow * (ncol // 16), body, None)


NBUF = 4


def _spmm_body(h_hbm, src_hbm, dst_hbm, out_hbm, idx_s, idx_d, bufs, zbuf,
               acc, sem_g, sem_s):
    c = lax.axis_index("c")
    s = lax.axis_index("s")

    pltpu.sync_copy(src_hbm.at[c, s], idx_s)
    pltpu.sync_copy(dst_hbm.at[s], idx_d)

    for b in range(NBUF):
        pltpu.async_copy(h_hbm.at[idx_s.at[b]], bufs[b], sem_g)

    _zero_vmem(zbuf, Z, FH)
    for t in range((NZ + NSUB - 1) // NSUB):
        j = s + NSUB * t

        @pl.when(j < NZ)
        def _():
            pltpu.sync_copy(zbuf, acc.at[pl.ds(j * Z, Z)])
    plsc.subcore_barrier()

    def chunk(i, carry):
        for b in range(NBUF):
            k = NBUF * i + b
            pltpu.make_async_copy(h_hbm.at[idx_s.at[k]], bufs[b],
                                  sem_g).wait()
            pltpu.async_copy(bufs[b], acc.at[idx_d.at[k]], sem_s, add=True)
        for b in range(NBUF):
            k = NBUF * i + b
            pltpu.make_async_copy(bufs[b], acc.at[idx_d.at[k]], sem_s).wait()

            @pl.when(k + NBUF < NCHUNK)
            def _():
                pltpu.async_copy(h_hbm.at[idx_s.at[k + NBUF]], bufs[b], sem_g)
        return carry
    lax.fori_loop(0, NCHUNK // NBUF, chunk, None)

    plsc.subcore_barrier()
    for t in range((NZ + NSUB - 1) // NSUB):
        j = s + NSUB * t

        @pl.when(j < NZ)
        def _():
            pltpu.sync_copy(acc.at[pl.ds(j * Z, Z)],
                            out_hbm.at[c, pl.ds(j * Z, Z)])


_sc_spmm = pl.kernel(
    _spmm_body,
    out_type=jax.ShapeDtypeStruct((NCORES, N, FH), jnp.float32),
    mesh=_mesh,
    scratch_types=[
        pltpu.VMEM((NCHUNK, C), jnp.int32),
        pltpu.VMEM((NCHUNK, C), jnp.int32),
        tuple(pltpu.VMEM((C, FH), jnp.float32) for _ in range(NBUF)),
        pltpu.VMEM((Z, FH), jnp.float32),
        pltpu.VMEM_SHARED((N, FH), jnp.float32),
        pltpu.SemaphoreType.DMA,
        pltpu.SemaphoreType.DMA,
    ],
    compiler_params=pltpu.CompilerParams(use_tc_tiling_on_sc=False))

NCHUNK_D = (E // (NCORES * NSUB)) // C
WIN = 8


def _cnt_body(dst_hbm, cnt_hbm, idx_d, ones, zbuf16, cnt_sh, sem):
    c = lax.axis_index("c")
    s = lax.axis_index("s")
    wid = c * NSUB + s
    pltpu.sync_copy(dst_hbm.at[wid], idx_d)

    _zero_vmem(zbuf16, Z, CW)

    def fill_ones(i, carry):
        ones[i, pl.ds(0, CW)] = jnp.ones((CW,), jnp.float32)
        return carry
    lax.fori_loop(0, C, fill_ones, None)
    for t in range((NZ + NSUB - 1) // NSUB):
        j = s + NSUB * t

        @pl.when(j < NZ)
        def _():
            pltpu.sync_copy(zbuf16, cnt_sh.at[pl.ds(j * Z, Z)])
    plsc.subcore_barrier()

    def chunk(k, carry):
        pltpu.async_copy(ones, cnt_sh.at[idx_d.at[k]], sem, add=True)

        @pl.when(k >= WIN)
        def _():
            pltpu.make_async_copy(ones, cnt_sh.at[idx_d.at[0]], sem).wait()
        return carry
    lax.fori_loop(0, NCHUNK_D, chunk, None)
    for _ in range(WIN):
        pltpu.make_async_copy(ones, cnt_sh.at[idx_d.at[0]], sem).wait()

    plsc.subcore_barrier()
    for t in range((NZ + NSUB - 1) // NSUB):
        j = s + NSUB * t

        @pl.when(j < NZ)
        def _():
            pltpu.sync_copy(cnt_sh.at[pl.ds(j * Z, Z)],
                            cnt_hbm.at[c, pl.ds(j * Z, Z)])


_sc_cnt = pl.kernel(
    _cnt_body,
    out_type=jax.ShapeDtypeStruct((NCORES, N, CW), jnp.float32),
    mesh=_mesh,
    scratch_types=[
        pltpu.VMEM((NCHUNK_D, C), jnp.int32),
        pltpu.VMEM((C, CW), jnp.float32),
        pltpu.VMEM((Z, CW), jnp.float32),
        pltpu.VMEM_SHARED((N, CW), jnp.float32),
        pltpu.SemaphoreType.DMA,
    ],
    compiler_params=pltpu.CompilerParams(use_tc_tiling_on_sc=False))

R = 2000


def _dots(mean, h, wl_ref, wr_ref, bl_ref):
    return (jnp.dot(mean, wl_ref[...], preferred_element_type=jnp.float32,
                    precision=lax.Precision.DEFAULT)
            + jnp.dot(h, wr_ref[...], preferred_element_type=jnp.float32,
                      precision=lax.Precision.DEFAULT)
            + bl_ref[...])


def _combine_body(p_ref, c_ref, h_ref, wl_ref, wr_ref, bl_ref, o_ref):
    cnt = jnp.maximum(c_ref[0, :, 0:1] + c_ref[1, :, 0:1], 1.0)
    mean = jnp.concatenate([p_ref[0], p_ref[1]], axis=1) / cnt
    h = jnp.concatenate([h_ref[0], h_ref[1]], axis=1)
    y = jnp.maximum(_dots(mean, h, wl_ref, wr_ref, bl_ref), 0.0)
    o_ref[0] = y[:, :FH]
    o_ref[1] = y[:, FH:]


def _final_body(p_ref, c_ref, h_ref, wl_ref, wr_ref, bl_ref, wc_ref, bc_ref,
                o_ref):
    cnt = jnp.maximum(c_ref[0, :, 0:1] + c_ref[1, :, 0:1], 1.0)
    mean = jnp.concatenate([p_ref[0], p_ref[1]], axis=1) / cnt
    h = jnp.concatenate([h_ref[0], h_ref[1]], axis=1)
    y = _dots(mean, h, wl_ref, wr_ref, bl_ref)
    o_ref[...] = (jnp.dot(y, wc_ref[...], preferred_element_type=jnp.float32,
                          precision=lax.Precision.DEFAULT)
                  + bc_ref[...])


_common_specs = [
    pl.BlockSpec((NCORES, R, FH), lambda i: (0, i, 0)),
    pl.BlockSpec((NCORES, R, CW), lambda i: (0, i, 0)),
    pl.BlockSpec((NCORES, R, FH), lambda i: (0, i, 0)),
    pl.BlockSpec((F, F), lambda i: (0, 0)),
    pl.BlockSpec((F, F), lambda i: (0, 0)),
    pl.BlockSpec((1, F), lambda i: (0, 0)),
]

_combine_relu = pl.pallas_call(
    _combine_body,
    grid=(N // R,),
    in_specs=_common_specs,
    out_specs=pl.BlockSpec((NCORES, R, FH), lambda i: (0, i, 0)),
    out_shape=jax.ShapeDtypeStruct((NCORES, N, FH), jnp.float32),
)

_combine_final = pl.pallas_call(
    _final_body,
    grid=(N // R,),
    in_specs=_common_specs + [
        pl.BlockSpec((F, NCLS), lambda i: (0, 0)),
        pl.BlockSpec((1, NCLS), lambda i: (0, 0)),
    ],
    out_specs=pl.BlockSpec((R, NCLS), lambda i: (i, 0)),
    out_shape=jax.ShapeDtypeStruct((N, NCLS), jnp.float32),
)


def kernel(x, edge_index, Wl0, bl0, Wr0, Wl1, bl1, Wr1, Wl2, bl2, Wr2, Wc,
           bc):
    src = edge_index[0].reshape(NSUB, NCHUNK, C)
    dst = edge_index[1].reshape(NSUB, NCHUNK, C)
    src2 = jnp.stack([src, src + N])
    dst_w = edge_index[1].reshape(NCORES * NSUB, NCHUNK_D, C)
    x2 = jnp.stack([x[:, :FH], x[:, FH:]])

    cnt = _sc_cnt(dst_w)
    p0 = _sc_spmm(x2.reshape(NCORES * N, FH), src2, dst)
    h1 = _combine_relu(p0, cnt, x2, Wl0, Wr0, bl0.reshape(1, F))
    p1 = _sc_spmm(h1.reshape(NCORES * N, FH), src2, dst)
    h2 = _combine_relu(p1, cnt, h1, Wl1, Wr1, bl1.reshape(1, F))
    p2 = _sc_spmm(h2.reshape(NCORES * N, FH), src2, dst)
    return _combine_final(p2, cnt, h2, Wl2, Wr2, bl2.reshape(1, F), Wc,
                          bc.reshape(1, NCLS))


# speedup vs baseline: 11.3058x; 1.0003x over previous
import functools

import jax
import jax.numpy as jnp
from jax import lax
from jax.experimental import pallas as pl
from jax.experimental.pallas import tpu as pltpu
from jax.experimental.pallas import tpu_sc as plsc

N = 10000
E = 320000
F = 128
FH = F // 2
NCLS = 64

NCORES = 2
NSUB = 16
EPS = E // NSUB
C = 125
NCHUNK = EPS // C
Z = 200
NZ = N // Z
CW = 16

_mesh = plsc.VectorSubcoreMesh(core_axis_name="c", subcore_axis_name="s")


def _zero_vmem(ref, nrow, ncol):
    def body(i, carry):
        ref[i // (ncol // 16), pl.ds((i % (ncol // 16)) * 16, 16)] = (
            jnp.zeros((16,), jnp.float32))
        return carry
    lax.fori_loop(0, nrow * (ncol // 16), body, None)


NBUF = 4


def _spmm_body(h_hbm, src_hbm, dst_hbm, out_hbm, idx_s, idx_d, bufs, zbuf,
               acc, sem_g, sem_s):
    c = lax.axis_index("c")
    s = lax.axis_index("s")

    pltpu.sync_copy(src_hbm.at[s], idx_s)
    pltpu.sync_copy(dst_hbm.at[s], idx_d)

    for b in range(NBUF):
        pltpu.async_copy(h_hbm.at[c].at[idx_s.at[b]], bufs[b], sem_g)

    _zero_vmem(zbuf, Z, FH)
    for t in range((NZ + NSUB - 1) // NSUB):
        j = s + NSUB * t

        @pl.when(j < NZ)
        def _():
            pltpu.sync_copy(zbuf, acc.at[pl.ds(j * Z, Z)])
    plsc.subcore_barrier()

    def chunk(i, carry):
        for b in range(NBUF):
            k = NBUF * i + b
            pltpu.make_async_copy(h_hbm.at[c].at[idx_s.at[k]], bufs[b],
                                  sem_g).wait()
            pltpu.async_copy(bufs[b], acc.at[idx_d.at[k]], sem_s, add=True)
        for b in range(NBUF):
            k = NBUF * i + b
            pltpu.make_async_copy(bufs[b], acc.at[idx_d.at[k]], sem_s).wait()

            @pl.when(k + NBUF < NCHUNK)
            def _():
                pltpu.async_copy(h_hbm.at[c].at[idx_s.at[k + NBUF]], bufs[b],
                                 sem_g)
        return carry
    lax.fori_loop(0, NCHUNK // NBUF, chunk, None)

    plsc.subcore_barrier()
    for t in range((NZ + NSUB - 1) // NSUB):
        j = s + NSUB * t

        @pl.when(j < NZ)
        def _():
            pltpu.sync_copy(acc.at[pl.ds(j * Z, Z)],
                            out_hbm.at[c, pl.ds(j * Z, Z)])


_sc_spmm = pl.kernel(
    _spmm_body,
    out_type=jax.ShapeDtypeStruct((NCORES, N, FH), jnp.float32),
    mesh=_mesh,
    scratch_types=[
        pltpu.VMEM((NCHUNK, C), jnp.int32),
        pltpu.VMEM((NCHUNK, C), jnp.int32),
        tuple(pltpu.VMEM((C, FH), jnp.float32) for _ in range(NBUF)),
        pltpu.VMEM((Z, FH), jnp.float32),
        pltpu.VMEM_SHARED((N, FH), jnp.float32),
        pltpu.SemaphoreType.DMA,
        pltpu.SemaphoreType.DMA,
    ],
    compiler_params=pltpu.CompilerParams(use_tc_tiling_on_sc=False))

NCHUNK_D = (E // (NCORES * NSUB)) // C
WIN = 8


def _cnt_body(dst_hbm, cnt_hbm, idx_d, ones, zbuf16, cnt_sh, sem):
    c = lax.axis_index("c")
    s = lax.axis_index("s")
    wid = c * NSUB + s
    pltpu.sync_copy(dst_hbm.at[wid], idx_d)

    _zero_vmem(zbuf16, Z, CW)

    def fill_ones(i, carry):
        ones[i, pl.ds(0, CW)] = jnp.ones((CW,), jnp.float32)
        return carry
    lax.fori_loop(0, C, fill_ones, None)
    for t in range((NZ + NSUB - 1) // NSUB):
        j = s + NSUB * t

        @pl.when(j < NZ)
        def _():
            pltpu.sync_copy(zbuf16, cnt_sh.at[pl.ds(j * Z, Z)])
    plsc.subcore_barrier()

    def chunk(k, carry):
        pltpu.async_copy(ones, cnt_sh.at[idx_d.at[k]], sem, add=True)

        @pl.when(k >= WIN)
        def _():
            pltpu.make_async_copy(ones, cnt_sh.at[idx_d.at[0]], sem).wait()
        return carry
    lax.fori_loop(0, NCHUNK_D, chunk, None)
    for _ in range(WIN):
        pltpu.make_async_copy(ones, cnt_sh.at[idx_d.at[0]], sem).wait()

    plsc.subcore_barrier()
    for t in range((NZ + NSUB - 1) // NSUB):
        j = s + NSUB * t

        @pl.when(j < NZ)
        def _():
            pltpu.sync_copy(cnt_sh.at[pl.ds(j * Z, Z)],
                            cnt_hbm.at[c, pl.ds(j * Z, Z)])


_sc_cnt = pl.kernel(
    _cnt_body,
    out_type=jax.ShapeDtypeStruct((NCORES, N, CW), jnp.float32),
    mesh=_mesh,
    scratch_types=[
        pltpu.VMEM((NCHUNK_D, C), jnp.int32),
        pltpu.VMEM((C, CW), jnp.float32),
        pltpu.VMEM((Z, CW), jnp.float32),
        pltpu.VMEM_SHARED((N, CW), jnp.float32),
        pltpu.SemaphoreType.DMA,
    ],
    compiler_params=pltpu.CompilerParams(use_tc_tiling_on_sc=False))

R = 2000


def _dots(mean, h, wl_ref, wr_ref, bl_ref):
    return (jnp.dot(mean, wl_ref[...], preferred_element_type=jnp.float32,
                    precision=lax.Precision.DEFAULT)
            + jnp.dot(h, wr_ref[...], preferred_element_type=jnp.float32,
                      precision=lax.Precision.DEFAULT)
            + bl_ref[...])


def _combine_body(p_ref, c_ref, h_ref, wl_ref, wr_ref, bl_ref, o_ref):
    cnt = jnp.maximum(c_ref[0, :, 0:1] + c_ref[1, :, 0:1], 1.0)
    mean = jnp.concatenate([p_ref[0], p_ref[1]], axis=1) / cnt
    h = jnp.concatenate([h_ref[0], h_ref[1]], axis=1)
    y = jnp.maximum(_dots(mean, h, wl_ref, wr_ref, bl_ref), 0.0)
    o_ref[0] = y[:, :FH]
    o_ref[1] = y[:, FH:]


def _final_body(p_ref, c_ref, h_ref, wl_ref, wr_ref, bl_ref, wc_ref, bc_ref,
                o_ref):
    cnt = jnp.maximum(c_ref[0, :, 0:1] + c_ref[1, :, 0:1], 1.0)
    mean = jnp.concatenate([p_ref[0], p_ref[1]], axis=1) / cnt
    h = jnp.concatenate([h_ref[0], h_ref[1]], axis=1)
    y = _dots(mean, h, wl_ref, wr_ref, bl_ref)
    o_ref[...] = (jnp.dot(y, wc_ref[...], preferred_element_type=jnp.float32,
                          precision=lax.Precision.DEFAULT)
                  + bc_ref[...])


_common_specs = [
    pl.BlockSpec((NCORES, R, FH), lambda i: (0, i, 0)),
    pl.BlockSpec((NCORES, R, CW), lambda i: (0, i, 0)),
    pl.BlockSpec((NCORES, R, FH), lambda i: (0, i, 0)),
    pl.BlockSpec((F, F), lambda i: (0, 0)),
    pl.BlockSpec((F, F), lambda i: (0, 0)),
    pl.BlockSpec((1, F), lambda i: (0, 0)),
]

_combine_relu = pl.pallas_call(
    _combine_body,
    grid=(N // R,),
    in_specs=_common_specs,
    out_specs=pl.BlockSpec((NCORES, R, FH), lambda i: (0, i, 0)),
    out_shape=jax.ShapeDtypeStruct((NCORES, N, FH), jnp.float32),
)

_combine_final = pl.pallas_call(
    _final_body,
    grid=(N // R,),
    in_specs=_common_specs + [
        pl.BlockSpec((F, NCLS), lambda i: (0, 0)),
        pl.BlockSpec((1, NCLS), lambda i: (0, 0)),
    ],
    out_specs=pl.BlockSpec((R, NCLS), lambda i: (i, 0)),
    out_shape=jax.ShapeDtypeStruct((N, NCLS), jnp.float32),
)


def kernel(x, edge_index, Wl0, bl0, Wr0, Wl1, bl1, Wr1, Wl2, bl2, Wr2, Wc,
           bc):
    src = edge_index[0].reshape(NSUB, NCHUNK, C)
    dst = edge_index[1].reshape(NSUB, NCHUNK, C)
    dst_w = edge_index[1].reshape(NCORES * NSUB, NCHUNK_D, C)
    x2 = jnp.stack([x[:, :FH], x[:, FH:]])

    cnt = _sc_cnt(dst_w)
    p0 = _sc_spmm(x2, src, dst)
    h1 = _combine_relu(p0, cnt, x2, Wl0, Wr0, bl0.reshape(1, F))
    p1 = _sc_spmm(h1, src, dst)
    h2 = _combine_relu(p1, cnt, h1, Wl1, Wr1, bl1.reshape(1, F))
    p2 = _sc_spmm(h2, src, dst)
    return _combine_final(p2, cnt, h2, Wl2, Wr2, bl2.reshape(1, F), Wc,
                          bc.reshape(1, NCLS))
